# baseline (speedup 1.0000x reference)
import functools

import jax
import jax.numpy as jnp
from jax import lax
from jax.experimental import pallas as pl
from jax.experimental.pallas import tpu as pltpu
from jax.experimental.pallas import tpu_sc as plsc

NE = 64
DM = 768
FF = 3072
NTOK = 2048
CAP = NTOK // NE

_HI = lax.Precision.HIGHEST


def _router_body(xf_ref, ws_ref, bs_ref, gidx_ref, src_ref):
    xf = xf_ref[...]
    ws = ws_ref[...]
    logits = jnp.dot(xf.astype(jnp.bfloat16), ws.astype(jnp.bfloat16),
                     preferred_element_type=jnp.float32) + bs_ref[...]
    m = jnp.max(logits, axis=1, keepdims=True)
    e_iota = lax.broadcasted_iota(jnp.int32, (NTOK, NE), 1)
    routes = jnp.min(jnp.where(logits == m, e_iota, NE), axis=1, keepdims=True)
    onehot = (e_iota == routes).astype(jnp.int32)
    c = onehot
    s = 1
    while s < NTOK:
        shifted = jnp.concatenate(
            [jnp.zeros((s, NE), jnp.int32), c[: NTOK - s, :]], axis=0)
        c = c + shifted
        s *= 2
    pos = jnp.sum(onehot * c, axis=1, keepdims=True) - 1
    kept = pos < CAP

    ohf = onehot.astype(jnp.float32)
    c_iota = lax.broadcasted_iota(jnp.int32, (NTOK, CAP), 1)
    pmask = (c_iota == pos).astype(jnp.float32)
    ids1 = (lax.broadcasted_iota(jnp.int32, (NTOK, CAP), 0) + 1
            ).astype(jnp.float32)
    dn = (((0,), (0,)), ((), ()))
    gsum = lax.dot_general(ohf, pmask * ids1, dn, precision=_HI)
    gsum_i = gsum.astype(jnp.int32)
    slot_iota = lax.broadcasted_iota(jnp.int32, (NE, CAP), 0) * CAP + \
        lax.broadcasted_iota(jnp.int32, (NE, CAP), 1)
    gidx_ref[...] = jnp.where(gsum_i > 0, gsum_i - 1, slot_iota)

    tok = lax.broadcasted_iota(jnp.int32, (NTOK, 1), 0)
    src_kept = routes * (2 * CAP) + pos
    src_drop = (tok // CAP) * (2 * CAP) + CAP + (tok % CAP)
    src_ref[...] = jnp.where(kept, src_kept, src_drop)


def _ffn_body(xe_ref, w1_ref, b1_ref, w2_ref, b2_ref, xf_ref, out_ref):
    f = pl.program_id(1)
    xe = xe_ref[...].astype(jnp.bfloat16)
    w1 = w1_ref[0].astype(jnp.bfloat16)
    h = jnp.dot(xe, w1, preferred_element_type=jnp.float32) + b1_ref[0]
    h = h * 0.5 * (1.0 + lax.erf(h / 1.41421))
    w2 = w2_ref[0].astype(jnp.bfloat16)
    o = jnp.dot(h.astype(jnp.bfloat16), w2,
                preferred_element_type=jnp.float32)

    @pl.when(f == 0)
    def _():
        out_ref[0:CAP, :] = o + b2_ref[0]
        out_ref[CAP:2 * CAP, :] = xf_ref[...]

    @pl.when(f > 0)
    def _():
        out_ref[0:CAP, :] += o


_NW = 32
_RPW = NTOK // _NW


@functools.lru_cache(maxsize=None)
def _make_sc_row_gather(name):
    mesh = plsc.VectorSubcoreMesh(core_axis_name="c", subcore_axis_name="s")

    @functools.partial(
        pl.kernel,
        mesh=mesh,
        out_type=jax.ShapeDtypeStruct((NTOK, DM), jnp.float32),
        scratch_types=[
            pltpu.VMEM((_RPW,), jnp.int32),
            pltpu.VMEM((_RPW, DM), jnp.float32),
            pltpu.SemaphoreType.DMA,
        ],
        name=name,
    )
    def sc_row_gather(table_hbm, idx_hbm, out_hbm, idx_v, rows_v, sem):
        wid = lax.axis_index("s") * 2 + lax.axis_index("c")
        base = wid * _RPW
        pltpu.sync_copy(idx_hbm.at[pl.ds(base, _RPW)], idx_v)
        pltpu.async_copy(table_hbm.at[idx_v], rows_v, sem).wait()
        pltpu.sync_copy(rows_v, out_hbm.at[pl.ds(base, _RPW)])

    return sc_row_gather


def kernel(x, W_switch, b_switch, W1, b1, W2, b2):
    B, S, Dm = x.shape
    xf = x.reshape(NTOK, DM)

    gidx, src = pl.pallas_call(
        _router_body,
        out_shape=(
            jax.ShapeDtypeStruct((NE, CAP), jnp.int32),
            jax.ShapeDtypeStruct((NTOK, 1), jnp.int32),
        ),
        name="tc_router",
    )(xf, W_switch, b_switch.reshape(1, NE))

    xe = _make_sc_row_gather("sc_dispatch")(xf, gidx.reshape(NTOK))

    big = pl.pallas_call(
        _ffn_body,
        grid=(NE, 4),
        in_specs=[
            pl.BlockSpec((CAP, DM), lambda e, f: (e, 0)),
            pl.BlockSpec((1, DM, FF // 4), lambda e, f: (e, 0, f)),
            pl.BlockSpec((1, 1, FF // 4), lambda e, f: (e, 0, f)),
            pl.BlockSpec((1, FF // 4, DM), lambda e, f: (e, f, 0)),
            pl.BlockSpec((1, 1, DM), lambda e, f: (e, 0, 0)),
            pl.BlockSpec((CAP, DM), lambda e, f: (e, 0)),
        ],
        out_specs=pl.BlockSpec((2 * CAP, DM), lambda e, f: (e, 0)),
        out_shape=jax.ShapeDtypeStruct((2 * NTOK, DM), jnp.float32),
        compiler_params=pltpu.CompilerParams(
            dimension_semantics=("arbitrary", "arbitrary"),
        ),
        name="tc_expert_ffn",
    )(xe, W1, b1.reshape(NE, 1, FF), W2, b2.reshape(NE, 1, DM), xf)

    out = _make_sc_row_gather("sc_combine")(big, src.reshape(NTOK))
    return out.reshape(B, S, Dm)


# speedup vs baseline: 1.1346x; 1.1346x over previous
import functools

import jax
import jax.numpy as jnp
from jax import lax
from jax.experimental import pallas as pl
from jax.experimental.pallas import tpu as pltpu
from jax.experimental.pallas import tpu_sc as plsc

NE = 64
DM = 768
FF = 3072
NTOK = 2048
CAP = NTOK // NE

_HI = lax.Precision.HIGHEST


def _router_body(xf_ref, ws_ref, bs_ref, gidx_ref, src_ref, packed_ref):
    xf = xf_ref[...]
    ws = ws_ref[...]
    logits = jnp.dot(xf.astype(jnp.bfloat16), ws.astype(jnp.bfloat16),
                     preferred_element_type=jnp.float32) + bs_ref[...]
    m = jnp.max(logits, axis=1, keepdims=True)
    e_iota = lax.broadcasted_iota(jnp.int32, (NTOK, NE), 1)
    routes = jnp.min(jnp.where(logits == m, e_iota, NE), axis=1, keepdims=True)
    onehot = (e_iota == routes).astype(jnp.int32)
    c = onehot
    s = 1
    while s < NTOK:
        shifted = jnp.concatenate(
            [jnp.zeros((s, NE), jnp.int32), c[: NTOK - s, :]], axis=0)
        c = c + shifted
        s *= 2
    pos = jnp.sum(onehot * c, axis=1, keepdims=True) - 1
    kept = pos < CAP

    ohf = onehot.astype(jnp.float32)
    c_iota = lax.broadcasted_iota(jnp.int32, (NTOK, CAP), 1)
    pmask = (c_iota == pos).astype(jnp.float32)
    ids1 = (lax.broadcasted_iota(jnp.int32, (NTOK, CAP), 0) + 1
            ).astype(jnp.float32)
    dn = (((0,), (0,)), ((), ()))
    gsum = lax.dot_general(ohf, pmask * ids1, dn, precision=_HI)
    gsum_i = gsum.astype(jnp.int32)
    slot_iota = lax.broadcasted_iota(jnp.int32, (NE, CAP), 0) * CAP + \
        lax.broadcasted_iota(jnp.int32, (NE, CAP), 1)
    gidx_ref[...] = jnp.where(gsum_i > 0, gsum_i - 1, slot_iota)

    tok = lax.broadcasted_iota(jnp.int32, (NTOK, 1), 0)
    src_kept = routes * (2 * CAP) + pos
    src_drop = (tok // CAP) * (2 * CAP) + CAP + (tok % CAP)
    src_ref[...] = jnp.where(kept, src_kept, src_drop)

    u = lax.bitcast_convert_type(xf, jnp.int32)
    r = jnp.right_shift(
        u + 0x7FFF + jnp.bitwise_and(jnp.right_shift(u, 16), 1), 16)
    lo = jnp.bitwise_and(r[:, : DM // 2], 0xFFFF)
    hi = jnp.left_shift(r[:, DM // 2:], 16)
    packed_ref[...] = jnp.bitwise_or(hi, lo)


def _ffn_body(xe_ref, w1_ref, b1_ref, w2_ref, b2_ref, xf_ref, out_ref):
    f = pl.program_id(1)
    p = xe_ref[...]
    xlo = lax.bitcast_convert_type(
        jnp.left_shift(p, 16), jnp.float32).astype(jnp.bfloat16)
    xhi = lax.bitcast_convert_type(
        jnp.bitwise_and(p, jnp.int32(-65536)), jnp.float32).astype(jnp.bfloat16)
    w1 = w1_ref[0].astype(jnp.bfloat16)
    h = (jnp.dot(xlo, w1[: DM // 2], preferred_element_type=jnp.float32)
         + jnp.dot(xhi, w1[DM // 2:], preferred_element_type=jnp.float32)
         + b1_ref[0])
    h = h * 0.5 * (1.0 + lax.erf(h / 1.41421))
    w2 = w2_ref[0].astype(jnp.bfloat16)
    o = jnp.dot(h.astype(jnp.bfloat16), w2,
                preferred_element_type=jnp.float32)

    @pl.when(f == 0)
    def _():
        out_ref[0:CAP, :] = o + b2_ref[0]
        out_ref[CAP:2 * CAP, :] = xf_ref[...]

    @pl.when(f == 1)
    def _():
        out_ref[0:CAP, :] += o


_NW = 32
_RPW = NTOK // _NW


@functools.lru_cache(maxsize=None)
def _make_sc_row_gather(name, cols, dtype):
    mesh = plsc.VectorSubcoreMesh(core_axis_name="c", subcore_axis_name="s")

    @functools.partial(
        pl.kernel,
        mesh=mesh,
        out_type=jax.ShapeDtypeStruct((NTOK, cols), dtype),
        scratch_types=[
            pltpu.VMEM((_RPW,), jnp.int32),
            pltpu.VMEM((_RPW, cols), dtype),
            pltpu.SemaphoreType.DMA,
        ],
        name=name,
    )
    def sc_row_gather(table_hbm, idx_hbm, out_hbm, idx_v, rows_v, sem):
        wid = lax.axis_index("s") * 2 + lax.axis_index("c")
        base = wid * _RPW
        pltpu.sync_copy(idx_hbm.at[pl.ds(base, _RPW)], idx_v)
        pltpu.async_copy(table_hbm.at[idx_v], rows_v, sem).wait()
        pltpu.sync_copy(rows_v, out_hbm.at[pl.ds(base, _RPW)])

    return sc_row_gather


def kernel(x, W_switch, b_switch, W1, b1, W2, b2):
    B, S, Dm = x.shape
    xf = x.reshape(NTOK, DM)

    gidx, src, packed = pl.pallas_call(
        _router_body,
        out_shape=(
            jax.ShapeDtypeStruct((NE, CAP), jnp.int32),
            jax.ShapeDtypeStruct((NTOK, 1), jnp.int32),
            jax.ShapeDtypeStruct((NTOK, DM // 2), jnp.int32),
        ),
        name="tc_router",
    )(xf, W_switch, b_switch.reshape(1, NE))

    xe = _make_sc_row_gather("sc_dispatch", DM // 2, jnp.int32)(
        packed, gidx.reshape(NTOK))

    big = pl.pallas_call(
        _ffn_body,
        grid=(NE, 2),
        in_specs=[
            pl.BlockSpec((CAP, DM // 2), lambda e, f: (e, 0)),
            pl.BlockSpec((1, DM, FF // 2), lambda e, f: (e, 0, f)),
            pl.BlockSpec((1, 1, FF // 2), lambda e, f: (e, 0, f)),
            pl.BlockSpec((1, FF // 2, DM), lambda e, f: (e, f, 0)),
            pl.BlockSpec((1, 1, DM), lambda e, f: (e, 0, 0)),
            pl.BlockSpec((CAP, DM), lambda e, f: (e, 0)),
        ],
        out_specs=pl.BlockSpec((2 * CAP, DM), lambda e, f: (e, 0)),
        out_shape=jax.ShapeDtypeStruct((2 * NTOK, DM), jnp.float32),
        compiler_params=pltpu.CompilerParams(
            dimension_semantics=("arbitrary", "arbitrary"),
        ),
        name="tc_expert_ffn",
    )(xe, W1, b1.reshape(NE, 1, FF), W2, b2.reshape(NE, 1, DM), xf)

    out = _make_sc_row_gather("sc_combine", DM, jnp.float32)(
        big, src.reshape(NTOK))
    return out.reshape(B, S, Dm)
